# Initial kernel scaffold; baseline (speedup 1.0000x reference)
import functools

import jax
import jax.numpy as jnp
from jax import lax
from jax.experimental import pallas as pl
from jax.experimental.pallas import tpu as pltpu
from jax.experimental.pallas import tpu_sc as plsc

N = 10000
E = 320000
D = 128
NPAD = 10240
NC = 2
NS = 16
NW = NC * NS
EPW = E // NW
CH = 128
NCHUNK = 80
EPW_PAD = NCHUNK * CH
DUMP = N


def _sc_mesh():
    return plsc.VectorSubcoreMesh(
        core_axis_name="c", subcore_axis_name="s", num_cores=NC, num_subcores=NS
    )


@functools.partial(
    pl.kernel,
    out_type=jax.ShapeDtypeStruct((4, NPAD), jnp.float32),
    mesh=_sc_mesh(),
    scratch_types=[
        pltpu.VMEM((NCHUNK, CH), jnp.int32),
        pltpu.VMEM((NCHUNK, CH), jnp.int32),
        pltpu.VMEM((NCHUNK, CH), jnp.int32),
        pltpu.VMEM((NCHUNK, CH), jnp.int32),
        pltpu.VMEM((NCHUNK, CH), jnp.float32),
        pltpu.VMEM((NCHUNK, CH), jnp.float32),
        pltpu.VMEM_SHARED((NPAD,), jnp.float32),
        pltpu.VMEM_SHARED((NPAD,), jnp.float32),
        pltpu.VMEM_SHARED((NPAD,), jnp.float32),
        pltpu.VMEM_SHARED((NPAD,), jnp.float32),
    ],
)
def _sc_segment_weights(
    dstp_hbm, srcp_hbm, dstn_hbm, srcn_hbm, ones_hbm, zeros_hbm,
    out_hbm,
    dstp_v, srcp_v, dstn_v, srcn_v, ones_v, w_v,
    cntp_sh, cntn_sh, sp_sh, sn_sh,
):
    c = lax.axis_index("c")
    s = lax.axis_index("s")
    wid = s * NC + c

    pltpu.sync_copy(dstp_hbm.at[wid], dstp_v)
    pltpu.sync_copy(srcp_hbm.at[wid], srcp_v)
    pltpu.sync_copy(dstn_hbm.at[wid], dstn_v)
    pltpu.sync_copy(srcn_hbm.at[wid], srcn_v)
    pltpu.sync_copy(ones_hbm, ones_v)

    @pl.when(s == 0)
    def _():
        pltpu.sync_copy(zeros_hbm, cntp_sh)
        pltpu.sync_copy(zeros_hbm, cntn_sh)
        pltpu.sync_copy(zeros_hbm, sp_sh)
        pltpu.sync_copy(zeros_hbm, sn_sh)

    plsc.subcore_barrier()

    pltpu.sync_copy(ones_v, cntp_sh.at[dstp_v], add=True)
    pltpu.sync_copy(ones_v, cntn_sh.at[dstn_v], add=True)

    plsc.subcore_barrier()

    def _relation(dst_v, src_v, cnt_sh, s_sh):
        pltpu.sync_copy(cnt_sh.at[dst_v], w_v)

        def _recip_row(r, carry):
            for t in range(CH // 16):
                sl = pl.ds(t * 16, 16)
                w_v[r, sl] = 1.0 / w_v[r, sl]
            return carry

        lax.fori_loop(0, NCHUNK, _recip_row, 0)
        pltpu.sync_copy(w_v, s_sh.at[src_v], add=True)

    _relation(dstp_v, srcp_v, cntp_sh, sp_sh)
    _relation(dstn_v, srcn_v, cntn_sh, sn_sh)

    plsc.subcore_barrier()

    @pl.when(s == 0)
    def _():
        pltpu.sync_copy(sp_sh, out_hbm.at[c])
        pltpu.sync_copy(sn_sh, out_hbm.at[2 + c])


BN = 2048
NSTEPS = NPAD // BN


def _tc_body(v_ref, x_ref, wlp_ref, wln_ref, wrp_ref, wrn_ref,
             blp_ref, bln_ref, out_ref, acc_ref):
    k = pl.program_id(0)

    @pl.when(k == 0)
    def _():
        acc_ref[...] = jnp.zeros_like(acc_ref)

    acc_ref[...] += jnp.dot(
        v_ref[...], x_ref[...], preferred_element_type=jnp.float32
    )

    @pl.when(k == NSTEPS - 1)
    def _():
        y = acc_ref[...]
        sp = y[0:1] + y[1:2]
        sn = y[2:3] + y[3:4]
        xs = y[4:5]
        r = (
            jnp.dot(sp, wlp_ref[...], preferred_element_type=jnp.float32)
            + jnp.dot(sn, wln_ref[...], preferred_element_type=jnp.float32)
            + jnp.dot(xs, wrp_ref[...], preferred_element_type=jnp.float32)
            + jnp.dot(xs, wrn_ref[...], preferred_element_type=jnp.float32)
        ) * (1.0 / N) + blp_ref[...] + bln_ref[...]
        out_ref[...] = jnp.broadcast_to(r, (8, 128))


_tc_combine = pl.pallas_call(
    _tc_body,
    grid=(NSTEPS,),
    in_specs=[
        pl.BlockSpec((8, BN), lambda k: (0, k)),
        pl.BlockSpec((BN, D), lambda k: (k, 0)),
        pl.BlockSpec((D, D), lambda k: (0, 0)),
        pl.BlockSpec((D, D), lambda k: (0, 0)),
        pl.BlockSpec((D, D), lambda k: (0, 0)),
        pl.BlockSpec((D, D), lambda k: (0, 0)),
        pl.BlockSpec((1, D), lambda k: (0, 0)),
        pl.BlockSpec((1, D), lambda k: (0, 0)),
    ],
    out_specs=pl.BlockSpec((8, D), lambda k: (0, 0)),
    out_shape=jax.ShapeDtypeStruct((8, D), jnp.float32),
    scratch_shapes=[pltpu.VMEM((8, D), jnp.float32)],
)


def _prep_edges(row):
    r = row.astype(jnp.int32).reshape(NW, EPW)
    pad = jnp.full((NW, EPW_PAD - EPW), DUMP, dtype=jnp.int32)
    return jnp.concatenate([r, pad], axis=1).reshape(NW, NCHUNK, CH)


@jax.jit
def kernel(x, edge_index_pos, edge_index_neg,
           Wl_pos, bl_pos, Wr_pos, Wl_neg, bl_neg, Wr_neg):
    xpad = jnp.pad(x, ((0, NPAD - N), (0, 0)))
    dstp = _prep_edges(edge_index_pos[1])
    srcp = _prep_edges(edge_index_pos[0])
    dstn = _prep_edges(edge_index_neg[1])
    srcn = _prep_edges(edge_index_neg[0])
    ones_vals = jnp.ones((NCHUNK, CH), jnp.float32)
    zeros_np = jnp.zeros((NPAD,), jnp.float32)

    s4 = _sc_segment_weights(dstp, srcp, dstn, srcn, ones_vals, zeros_np)

    v = jnp.concatenate(
        [s4, jnp.ones((1, NPAD), jnp.float32), jnp.zeros((3, NPAD), jnp.float32)],
        axis=0,
    )
    out8 = _tc_combine(
        v, xpad, Wl_pos.T, Wl_neg.T, Wr_pos.T, Wr_neg.T,
        bl_pos.reshape(1, D), bl_neg.reshape(1, D),
    )
    return out8[0:1]


# speedup vs baseline: 39.1990x; 39.1990x over previous
import functools

import jax
import jax.numpy as jnp
from jax import lax
from jax.experimental import pallas as pl
from jax.experimental.pallas import tpu as pltpu
from jax.experimental.pallas import tpu_sc as plsc

N = 10000
E = 320000
D = 128
NPAD = 10240
NC = 2
NS = 16
EPW = E // NS


def _sc_mesh():
    return plsc.VectorSubcoreMesh(
        core_axis_name="c", subcore_axis_name="s", num_cores=NC, num_subcores=NS
    )


@functools.partial(
    pl.kernel,
    out_type=jax.ShapeDtypeStruct((NC, NPAD), jnp.float32),
    mesh=_sc_mesh(),
    scratch_types=[
        pltpu.VMEM((EPW,), jnp.int32),
        pltpu.VMEM((EPW,), jnp.int32),
        pltpu.VMEM((EPW,), jnp.float32),
        pltpu.VMEM((EPW,), jnp.float32),
        pltpu.VMEM_SHARED((NPAD,), jnp.float32),
        pltpu.VMEM_SHARED((NPAD,), jnp.float32),
    ],
)
def _sc_segment_weights(
    dst_hbm, src_hbm, ones_hbm, zeros_hbm,
    out_hbm,
    dst_v, src_v, ones_v, w_v,
    cnt_sh, s_sh,
):
    c = lax.axis_index("c")
    s = lax.axis_index("s")

    pltpu.sync_copy(dst_hbm.at[c].at[s], dst_v)
    pltpu.sync_copy(src_hbm.at[c].at[s], src_v)
    pltpu.sync_copy(ones_hbm, ones_v)

    @pl.when(s == 0)
    def _():
        pltpu.sync_copy(zeros_hbm, cnt_sh)
        pltpu.sync_copy(zeros_hbm, s_sh)

    plsc.subcore_barrier()

    pltpu.sync_copy(ones_v, cnt_sh.at[dst_v], add=True)

    plsc.subcore_barrier()

    pltpu.sync_copy(cnt_sh.at[dst_v], w_v)

    def _recip(i, carry):
        for t in range(10):
            sl = pl.ds(i * 160 + t * 16, 16)
            w_v[sl] = 1.0 / w_v[sl]
        return carry

    lax.fori_loop(0, EPW // 160, _recip, 0)
    pltpu.sync_copy(w_v, s_sh.at[src_v], add=True)

    plsc.subcore_barrier()

    @pl.when(s == 0)
    def _():
        pltpu.sync_copy(s_sh, out_hbm.at[c])


BN = 2048
NSTEPS = NPAD // BN


def _tc_body(v_ref, x_ref, wlp_ref, wln_ref, wrp_ref, wrn_ref,
             blp_ref, bln_ref, out_ref, acc_ref):
    k = pl.program_id(0)

    @pl.when(k == 0)
    def _():
        acc_ref[...] = jnp.zeros_like(acc_ref)

    acc_ref[...] += jnp.dot(
        v_ref[...], x_ref[...], preferred_element_type=jnp.float32
    )

    @pl.when(k == NSTEPS - 1)
    def _():
        y = acc_ref[...]
        sp = y[0:1]
        sn = y[1:2]
        xs = y[2:3]
        r = (
            jnp.dot(sp, wlp_ref[...], preferred_element_type=jnp.float32)
            + jnp.dot(sn, wln_ref[...], preferred_element_type=jnp.float32)
            + jnp.dot(xs, wrp_ref[...], preferred_element_type=jnp.float32)
            + jnp.dot(xs, wrn_ref[...], preferred_element_type=jnp.float32)
        ) * (1.0 / N) + blp_ref[...] + bln_ref[...]
        out_ref[...] = jnp.broadcast_to(r, (8, 128))


_tc_combine = pl.pallas_call(
    _tc_body,
    grid=(NSTEPS,),
    in_specs=[
        pl.BlockSpec((8, BN), lambda k: (0, k)),
        pl.BlockSpec((BN, D), lambda k: (k, 0)),
        pl.BlockSpec((D, D), lambda k: (0, 0)),
        pl.BlockSpec((D, D), lambda k: (0, 0)),
        pl.BlockSpec((D, D), lambda k: (0, 0)),
        pl.BlockSpec((D, D), lambda k: (0, 0)),
        pl.BlockSpec((1, D), lambda k: (0, 0)),
        pl.BlockSpec((1, D), lambda k: (0, 0)),
    ],
    out_specs=pl.BlockSpec((8, D), lambda k: (0, 0)),
    out_shape=jax.ShapeDtypeStruct((8, D), jnp.float32),
    scratch_shapes=[pltpu.VMEM((8, D), jnp.float32)],
)


@jax.jit
def kernel(x, edge_index_pos, edge_index_neg,
           Wl_pos, bl_pos, Wr_pos, Wl_neg, bl_neg, Wr_neg):
    xpad = jnp.pad(x, ((0, NPAD - N), (0, 0)))
    dst_all = jnp.stack(
        [edge_index_pos[1].astype(jnp.int32).reshape(NS, EPW),
         edge_index_neg[1].astype(jnp.int32).reshape(NS, EPW)]
    )
    src_all = jnp.stack(
        [edge_index_pos[0].astype(jnp.int32).reshape(NS, EPW),
         edge_index_neg[0].astype(jnp.int32).reshape(NS, EPW)]
    )
    ones_vals = jnp.ones((EPW,), jnp.float32)
    zeros_np = jnp.zeros((NPAD,), jnp.float32)

    s2 = _sc_segment_weights(dst_all, src_all, ones_vals, zeros_np)

    v = jnp.concatenate(
        [s2, jnp.ones((1, NPAD), jnp.float32), jnp.zeros((5, NPAD), jnp.float32)],
        axis=0,
    )
    out8 = _tc_combine(
        v, xpad, Wl_pos.T, Wl_neg.T, Wr_pos.T, Wr_neg.T,
        bl_pos.reshape(1, D), bl_neg.reshape(1, D),
    )
    return out8[0:1]
